# baseline (speedup 1.0000x reference)
import functools

import jax
import jax.numpy as jnp
from jax import lax
from jax.experimental import pallas as pl
from jax.experimental.pallas import tpu as pltpu
from jax.experimental.pallas import tpu_sc as plsc

_NC = 2
_NS = 16
_NW = _NC * _NS

_C = 128
_K = 8


def _mm_body(h_ref, w_ref, o_ref):
    o_ref[0] = jnp.dot(h_ref[...], w_ref[0], preferred_element_type=jnp.float32)


def _all_transforms(h, weight, bn):
    n, f_in = h.shape
    r, _, f_out = weight.shape
    del bn
    return pl.pallas_call(
        _mm_body,
        grid=(r,),
        in_specs=[
            pl.BlockSpec((n, f_in), lambda ri: (0, 0)),
            pl.BlockSpec((1, f_in, f_out), lambda ri: (ri, 0, 0)),
        ],
        out_specs=pl.BlockSpec((1, n, f_out), lambda ri: (ri, 0, 0)),
        out_shape=jax.ShapeDtypeStruct((r, n, f_out), jnp.float32),
    )(h, weight)


def _add_body(p_ref, o_ref):
    o_ref[...] = p_ref[0] + p_ref[1]


def _sum_partials(partials, n, bn):
    _, _, f = partials.shape
    return pl.pallas_call(
        _add_body,
        grid=(n // bn,),
        in_specs=[pl.BlockSpec((2, bn, f), lambda i: (0, i, 0))],
        out_specs=pl.BlockSpec((bn, f), lambda i: (i, 0)),
        out_shape=jax.ShapeDtypeStruct((n, f), jnp.float32),
    )(partials)


def _make_sc_gather_scatter(n_pad, f_out, nb, k, c):
    rows_per_tile = n_pad // _NS
    mesh = plsc.VectorSubcoreMesh(core_axis_name="c", subcore_axis_name="s")

    @functools.partial(
        pl.kernel,
        mesh=mesh,
        out_type=jax.ShapeDtypeStruct((_NC, n_pad, f_out), jnp.float32),
        scratch_types=[
            pltpu.VMEM((k, c), jnp.int32),
            pltpu.VMEM((k, c), jnp.int32),
            pltpu.VMEM((k, c), jnp.int32),
            pltpu.VMEM((k, c), jnp.int32),
            pltpu.VMEM((c, f_out), jnp.float32),
            pltpu.VMEM((c, f_out), jnp.float32),
            pltpu.VMEM_SHARED((n_pad, f_out), jnp.float32),
            pltpu.SemaphoreType.DMA,
            pltpu.SemaphoreType.DMA,
            pltpu.SemaphoreType.DMA,
            pltpu.SemaphoreType.DMA,
            pltpu.SemaphoreType.DMA,
            pltpu.SemaphoreType.DMA,
        ],
    )
    def sc_kernel(all_t_hbm, fidx_hbm, didx_hbm, zeros_hbm, out_hbm,
                  fidx0, fidx1, didx0, didx1, rows0, rows1, acc_sh,
                  semi0, semi1, semr0, semr1, sems0, sems1):
        cid = lax.axis_index("c")
        sid = lax.axis_index("s")
        wid = cid * _NS + sid

        fbufs = (fidx0, fidx1)
        dbufs = (didx0, didx1)
        isems = (semi0, semi1)
        rbufs = (rows0, rows1)
        rsems = (semr0, semr1)
        ssems = (sems0, sems1)

        def start_idx_block(bi, p):
            pltpu.async_copy(fidx_hbm.at[wid, bi], fbufs[p], isems[p])
            pltpu.async_copy(didx_hbm.at[wid, bi], dbufs[p], isems[p])

        def wait_idx_block(p):
            pltpu.make_async_copy(fidx_hbm.at[0, 0],
                                  fbufs[p], isems[p]).wait()
            pltpu.make_async_copy(didx_hbm.at[0, 0],
                                  dbufs[p], isems[p]).wait()

        row0 = sid * rows_per_tile
        start_idx_block(0, 0)
        pltpu.sync_copy(zeros_hbm.at[pl.ds(row0, rows_per_tile)],
                        acc_sh.at[pl.ds(row0, rows_per_tile)])
        plsc.subcore_barrier()

        def block_body(bi, _):
            pb = lax.rem(bi, 2)
            for p in range(2):
                @pl.when(pb == p)
                def _():
                    wait_idx_block(p)

                    @pl.when(bi + 1 < nb)
                    def _():
                        start_idx_block(bi + 1, 1 - p)

                    pltpu.async_copy(all_t_hbm.at[fbufs[p].at[0]],
                                     rbufs[0], rsems[0])
                    for ki in range(k):
                        q = ki % 2
                        pltpu.make_async_copy(all_t_hbm.at[fbufs[p].at[ki]],
                                              rbufs[q], rsems[q]).wait()
                        pltpu.async_copy(rbufs[q], acc_sh.at[dbufs[p].at[ki]],
                                         ssems[q], add=True)
                        if ki + 1 < k:
                            if ki >= 1:
                                pltpu.make_async_copy(
                                    rbufs[1 - q], acc_sh.at[dbufs[p].at[ki]],
                                    ssems[1 - q]).wait()
                            pltpu.async_copy(
                                all_t_hbm.at[fbufs[p].at[ki + 1]],
                                rbufs[1 - q], rsems[1 - q])
                    for j in range(max(k - 2, 0), k):
                        pltpu.make_async_copy(rbufs[j % 2],
                                              acc_sh.at[dbufs[p].at[0]],
                                              ssems[j % 2]).wait()
            return 0

        lax.fori_loop(0, nb, block_body, 0)
        plsc.subcore_barrier()

        pltpu.sync_copy(acc_sh.at[pl.ds(row0, rows_per_tile)],
                        out_hbm.at[cid, pl.ds(row0, rows_per_tile)])

    return sc_kernel


def kernel(h, edge_index, rel_type, weight):
    n, f_in = h.shape
    r, _, f_out = weight.shape
    e = edge_index.shape[1]

    src = edge_index[0]
    dst = edge_index[1]
    flat_idx = rel_type * n + src

    rows_per_tile = ((n + 1 + _NS - 1) // _NS + 7) // 8 * 8
    n_pad = rows_per_tile * _NS

    per_block = _NW * _C * _K
    e_pad = ((e + per_block - 1) // per_block) * per_block
    if e_pad != e:
        pad = e_pad - e
        pad_gather = (jnp.arange(pad, dtype=jnp.int32) * 16) % (r * n)
        pad_dst = n + (jnp.arange(pad, dtype=jnp.int32) % (n_pad - n))
        flat_idx = jnp.concatenate([flat_idx, pad_gather])
        dst = jnp.concatenate([dst, pad_dst.astype(jnp.int32)])
    g = e_pad // (_NW * _C)
    nb = g // _K

    all_t = _all_transforms(h, weight, bn=2000)
    all_t_flat = all_t.reshape(r * n, f_out)

    fidx = flat_idx.reshape(_NW, nb, _K, _C)
    didx = dst.reshape(_NW, nb, _K, _C)
    zeros = jnp.zeros((n_pad, f_out), jnp.float32)

    sc_fn = _make_sc_gather_scatter(n_pad, f_out, nb, _K, _C)
    partials = sc_fn(all_t_flat, fidx, didx, zeros)

    return _sum_partials(partials, n, bn=1000)


# speedup vs baseline: 1.2186x; 1.2186x over previous
import functools

import jax
import jax.numpy as jnp
from jax import lax
from jax.experimental import pallas as pl
from jax.experimental.pallas import tpu as pltpu
from jax.experimental.pallas import tpu_sc as plsc

_NC = 2
_NS = 16
_NW = _NC * _NS

_C = 80
_K = 25


def _mm_body(h_ref, w_ref, o_ref):
    o_ref[0] = jnp.dot(h_ref[...], w_ref[0], preferred_element_type=jnp.float32)


def _all_transforms(h, weight, bn):
    n, f_in = h.shape
    r, _, f_out = weight.shape
    del bn
    return pl.pallas_call(
        _mm_body,
        grid=(r,),
        in_specs=[
            pl.BlockSpec((n, f_in), lambda ri: (0, 0)),
            pl.BlockSpec((1, f_in, f_out), lambda ri: (ri, 0, 0)),
        ],
        out_specs=pl.BlockSpec((1, n, f_out), lambda ri: (ri, 0, 0)),
        out_shape=jax.ShapeDtypeStruct((r, n, f_out), jnp.float32),
    )(h, weight)


def _add_body(p_ref, o_ref):
    o_ref[...] = p_ref[0] + p_ref[1]


def _sum_partials(partials, n, bn):
    _, _, f = partials.shape
    return pl.pallas_call(
        _add_body,
        grid=(n // bn,),
        in_specs=[pl.BlockSpec((2, bn, f), lambda i: (0, i, 0))],
        out_specs=pl.BlockSpec((bn, f), lambda i: (i, 0)),
        out_shape=jax.ShapeDtypeStruct((n, f), jnp.float32),
    )(partials)


def _make_sc_gather_scatter(n_pad, f_out, nb, k, c):
    rows_per_tile = n_pad // _NS
    mesh = plsc.VectorSubcoreMesh(core_axis_name="c", subcore_axis_name="s")

    @functools.partial(
        pl.kernel,
        mesh=mesh,
        out_type=jax.ShapeDtypeStruct((_NC, n_pad, f_out), jnp.float32),
        scratch_types=[
            pltpu.VMEM((k * c,), jnp.int32),
            pltpu.VMEM((k * c,), jnp.int32),
            pltpu.VMEM((k, c), jnp.int32),
            pltpu.VMEM((k, c), jnp.int32),
            pltpu.VMEM((c, f_out), jnp.float32),
            pltpu.VMEM((c, f_out), jnp.float32),
            pltpu.VMEM((c, f_out), jnp.float32),
            pltpu.VMEM_SHARED((n_pad, f_out), jnp.float32),
            pltpu.SemaphoreType.DMA,
            pltpu.SemaphoreType.DMA,
            pltpu.SemaphoreType.DMA,
            pltpu.SemaphoreType.DMA,
            pltpu.SemaphoreType.DMA,
            pltpu.SemaphoreType.DMA,
            pltpu.SemaphoreType.DMA,
            pltpu.SemaphoreType.DMA,
        ],
    )
    def sc_kernel(all_t_hbm, fidx_hbm, didx_hbm, out_hbm,
                  fidx0, fidx1, didx0, didx1, rows0, rows1, rows2, acc_sh,
                  semi0, semi1, semr0, semr1, semr2, sems0, sems1, sems2):
        cid = lax.axis_index("c")
        sid = lax.axis_index("s")
        wid = cid * _NS + sid

        fbufs = (fidx0, fidx1)
        dbufs = (didx0, didx1)
        isems = (semi0, semi1)
        rbufs = (rows0, rows1, rows2)
        rsems = (semr0, semr1, semr2)
        ssems = (sems0, sems1, sems2)

        ept = nb * k * c
        base = wid * ept

        def start_idx_block(bi, p):
            pltpu.async_copy(fidx_hbm.at[pl.ds(base + bi * k * c, k * c)],
                             fbufs[p], isems[p])
            pltpu.async_copy(didx_hbm.at[wid, bi], dbufs[p], isems[p])

        def wait_idx_block(p):
            pltpu.make_async_copy(fidx_hbm.at[pl.ds(0, k * c)],
                                  fbufs[p], isems[p]).wait()
            pltpu.make_async_copy(didx_hbm.at[0, 0],
                                  dbufs[p], isems[p]).wait()

        row0 = sid * rows_per_tile
        start_idx_block(0, 0)
        zvec = jnp.zeros((16,), jnp.float32)

        def zrow(i, _):
            for j in range(f_out // 16):
                rows0[i, pl.ds(j * 16, 16)] = zvec
            return 0

        lax.fori_loop(0, c, zrow, 0)
        nfull = rows_per_tile // c
        for i in range(nfull):
            pltpu.sync_copy(rows0, acc_sh.at[pl.ds(row0 + i * c, c)])
        rem = rows_per_tile - nfull * c
        if rem:
            pltpu.sync_copy(rows0.at[pl.ds(0, rem)],
                            acc_sh.at[pl.ds(row0 + nfull * c, rem)])
        plsc.subcore_barrier()

        def block_body(bi, _):
            pb = lax.rem(bi, 2)
            for p in range(2):
                @pl.when(pb == p)
                def _():
                    wait_idx_block(p)

                    @pl.when(bi + 1 < nb)
                    def _():
                        start_idx_block(bi + 1, 1 - p)

                    pltpu.async_copy(all_t_hbm.at[fbufs[p].at[pl.ds(0, c)]],
                                     rbufs[0], rsems[0])
                    pltpu.async_copy(all_t_hbm.at[fbufs[p].at[pl.ds(c, c)]],
                                     rbufs[1], rsems[1])
                    for ki in range(k):
                        q = ki % 3
                        pltpu.make_async_copy(all_t_hbm.at[fbufs[p].at[pl.ds(ki * c, c)]],
                                              rbufs[q], rsems[q]).wait()
                        pltpu.async_copy(rbufs[q], acc_sh.at[dbufs[p].at[ki]],
                                         ssems[q], add=True)
                        if 1 <= ki and ki + 2 < k:
                            qn = (ki + 2) % 3
                            pltpu.make_async_copy(
                                rbufs[qn], acc_sh.at[dbufs[p].at[ki]],
                                ssems[qn]).wait()
                        if ki + 2 < k:
                            pltpu.async_copy(
                                all_t_hbm.at[fbufs[p].at[pl.ds((ki + 2) * c, c)]],
                                rbufs[(ki + 2) % 3], rsems[(ki + 2) % 3])
                    for j in range(max(k - 3, 0), k):
                        pltpu.make_async_copy(rbufs[j % 3],
                                              acc_sh.at[dbufs[p].at[0]],
                                              ssems[j % 3]).wait()
            return 0

        lax.fori_loop(0, nb, block_body, 0)
        plsc.subcore_barrier()

        pltpu.sync_copy(acc_sh.at[pl.ds(row0, rows_per_tile)],
                        out_hbm.at[cid, pl.ds(row0, rows_per_tile)])

    return sc_kernel


def kernel(h, edge_index, rel_type, weight):
    n, f_in = h.shape
    r, _, f_out = weight.shape
    e = edge_index.shape[1]

    src = edge_index[0]
    dst = edge_index[1]
    flat_idx = rel_type * n + src

    rows_per_tile = ((n + 1 + _NS - 1) // _NS + 7) // 8 * 8
    n_pad = rows_per_tile * _NS

    per_block = _NW * _C * _K
    e_pad = ((e + per_block - 1) // per_block) * per_block
    if e_pad != e:
        pad = e_pad - e
        pad_gather = (jnp.arange(pad, dtype=jnp.int32) * 16) % (r * n)
        pad_dst = n + (jnp.arange(pad, dtype=jnp.int32) % (n_pad - n))
        flat_idx = jnp.concatenate([flat_idx, pad_gather])
        dst = jnp.concatenate([dst, pad_dst.astype(jnp.int32)])
    g = e_pad // (_NW * _C)
    nb = g // _K

    all_t = _all_transforms(h, weight, bn=2000)
    all_t_flat = all_t.reshape(r * n, f_out)

    didx = dst.reshape(_NW, nb, _K, _C)

    sc_fn = _make_sc_gather_scatter(n_pad, f_out, nb, _K, _C)
    partials = sc_fn(all_t_flat, flat_idx, didx)

    return _sum_partials(partials, n, bn=2000)


# speedup vs baseline: 1.2186x; 1.0000x over previous
import functools

import jax
import jax.numpy as jnp
from jax import lax
from jax.experimental import pallas as pl
from jax.experimental.pallas import tpu as pltpu
from jax.experimental.pallas import tpu_sc as plsc

_NC = 2
_NS = 16
_NW = _NC * _NS

_C = 80
_K = 25


def _mm_body(h_ref, w_ref, o_ref):
    o_ref[0] = jnp.dot(h_ref[...], w_ref[0], preferred_element_type=jnp.float32)


def _all_transforms(h, weight, bn):
    n, f_in = h.shape
    r, _, f_out = weight.shape
    del bn
    return pl.pallas_call(
        _mm_body,
        grid=(r,),
        in_specs=[
            pl.BlockSpec((n, f_in), lambda ri: (0, 0)),
            pl.BlockSpec((1, f_in, f_out), lambda ri: (ri, 0, 0)),
        ],
        out_specs=pl.BlockSpec((1, n, f_out), lambda ri: (ri, 0, 0)),
        out_shape=jax.ShapeDtypeStruct((r, n, f_out), jnp.float32),
    )(h, weight)


def _add_body(p_ref, o_ref):
    o_ref[...] = p_ref[0] + p_ref[1]


def _sum_partials(partials, n, bn):
    _, _, f = partials.shape
    return pl.pallas_call(
        _add_body,
        grid=(n // bn,),
        in_specs=[pl.BlockSpec((2, bn, f), lambda i: (0, i, 0))],
        out_specs=pl.BlockSpec((bn, f), lambda i: (i, 0)),
        out_shape=jax.ShapeDtypeStruct((n, f), jnp.float32),
    )(partials)


def _make_sc_gather_scatter(n_pad, f_out, nb, k, c):
    rows_per_tile = n_pad // _NS
    mesh = plsc.VectorSubcoreMesh(core_axis_name="c", subcore_axis_name="s")

    @functools.partial(
        pl.kernel,
        mesh=mesh,
        out_type=jax.ShapeDtypeStruct((_NC, n_pad, f_out), jnp.float32),
        scratch_types=[
            pltpu.VMEM((k * c,), jnp.int32),
            pltpu.VMEM((k * c,), jnp.int32),
            pltpu.VMEM((k * c,), jnp.int32),
            pltpu.VMEM((k * c,), jnp.int32),
            pltpu.VMEM((k, c), jnp.int32),
            pltpu.VMEM((k, c), jnp.int32),
            pltpu.VMEM((c, f_out), jnp.float32),
            pltpu.VMEM((c, f_out), jnp.float32),
            pltpu.VMEM((c, f_out), jnp.float32),
            pltpu.VMEM_SHARED((n_pad, f_out), jnp.float32),
            pltpu.SemaphoreType.DMA,
            pltpu.SemaphoreType.DMA,
            pltpu.SemaphoreType.DMA,
            pltpu.SemaphoreType.DMA,
            pltpu.SemaphoreType.DMA,
            pltpu.SemaphoreType.DMA,
            pltpu.SemaphoreType.DMA,
            pltpu.SemaphoreType.DMA,
        ],
    )
    def sc_kernel(all_t_hbm, fidx_hbm, didx_hbm, out_hbm,
                  fidx0, fidx1, didx0, didx1, didx2d0, didx2d1,
                  rows0, rows1, rows2, acc_sh,
                  semi0, semi1, semr0, semr1, semr2, sems0, sems1, sems2):
        cid = lax.axis_index("c")
        sid = lax.axis_index("s")
        wid = cid * _NS + sid

        fbufs = (fidx0, fidx1)
        dbufs1 = (didx0, didx1)
        dbufs = (didx2d0, didx2d1)
        isems = (semi0, semi1)
        rbufs = (rows0, rows1, rows2)
        rsems = (semr0, semr1, semr2)
        ssems = (sems0, sems1, sems2)

        ept = nb * k * c
        base = wid * ept

        def start_idx_block(bi, p):
            pltpu.async_copy(fidx_hbm.at[pl.ds(base + bi * k * c, k * c)],
                             fbufs[p], isems[p])
            pltpu.async_copy(didx_hbm.at[pl.ds(base + bi * k * c, k * c)],
                             dbufs1[p], isems[p])

        def wait_idx_block(p):
            pltpu.make_async_copy(fidx_hbm.at[pl.ds(0, k * c)],
                                  fbufs[p], isems[p]).wait()
            pltpu.make_async_copy(didx_hbm.at[pl.ds(0, k * c)],
                                  dbufs1[p], isems[p]).wait()

        row0 = sid * rows_per_tile
        start_idx_block(0, 0)
        zvec = jnp.zeros((16,), jnp.float32)

        def zrow(i, _):
            for j in range(f_out // 16):
                rows0[i, pl.ds(j * 16, 16)] = zvec
            return 0

        lax.fori_loop(0, c, zrow, 0)
        nfull = rows_per_tile // c
        for i in range(nfull):
            pltpu.sync_copy(rows0, acc_sh.at[pl.ds(row0 + i * c, c)])
        rem = rows_per_tile - nfull * c
        if rem:
            pltpu.sync_copy(rows0.at[pl.ds(0, rem)],
                            acc_sh.at[pl.ds(row0 + nfull * c, rem)])
        plsc.subcore_barrier()

        def block_body(bi, _):
            pb = lax.rem(bi, 2)
            for p in range(2):
                @pl.when(pb == p)
                def _():
                    wait_idx_block(p)

                    @pl.when(bi + 1 < nb)
                    def _():
                        start_idx_block(bi + 1, 1 - p)

                    def relay(ki2, _):
                        for j in range(c // 16):
                            dbufs[p][ki2, pl.ds(j * 16, 16)] = (
                                dbufs1[p][pl.ds(ki2 * c + j * 16, 16)])
                        return 0

                    lax.fori_loop(0, k, relay, 0)

                    pltpu.async_copy(all_t_hbm.at[fbufs[p].at[pl.ds(0, c)]],
                                     rbufs[0], rsems[0])
                    pltpu.async_copy(all_t_hbm.at[fbufs[p].at[pl.ds(c, c)]],
                                     rbufs[1], rsems[1])
                    for ki in range(k):
                        q = ki % 3
                        pltpu.make_async_copy(all_t_hbm.at[fbufs[p].at[pl.ds(ki * c, c)]],
                                              rbufs[q], rsems[q]).wait()
                        pltpu.async_copy(rbufs[q], acc_sh.at[dbufs[p].at[ki]],
                                         ssems[q], add=True)
                        if 1 <= ki and ki + 2 < k:
                            qn = (ki + 2) % 3
                            pltpu.make_async_copy(
                                rbufs[qn], acc_sh.at[dbufs[p].at[ki]],
                                ssems[qn]).wait()
                        if ki + 2 < k:
                            pltpu.async_copy(
                                all_t_hbm.at[fbufs[p].at[pl.ds((ki + 2) * c, c)]],
                                rbufs[(ki + 2) % 3], rsems[(ki + 2) % 3])
                    for j in range(max(k - 3, 0), k):
                        pltpu.make_async_copy(rbufs[j % 3],
                                              acc_sh.at[dbufs[p].at[0]],
                                              ssems[j % 3]).wait()
            return 0

        lax.fori_loop(0, nb, block_body, 0)
        plsc.subcore_barrier()

        pltpu.sync_copy(acc_sh.at[pl.ds(row0, rows_per_tile)],
                        out_hbm.at[cid, pl.ds(row0, rows_per_tile)])

    return sc_kernel


def kernel(h, edge_index, rel_type, weight):
    n, f_in = h.shape
    r, _, f_out = weight.shape
    e = edge_index.shape[1]

    src = edge_index[0]
    dst = edge_index[1]
    flat_idx = rel_type * n + src

    rows_per_tile = ((n + 1 + _NS - 1) // _NS + 7) // 8 * 8
    n_pad = rows_per_tile * _NS

    per_block = _NW * _C * _K
    e_pad = ((e + per_block - 1) // per_block) * per_block
    if e_pad != e:
        pad = e_pad - e
        pad_gather = (jnp.arange(pad, dtype=jnp.int32) * 16) % (r * n)
        pad_dst = n + (jnp.arange(pad, dtype=jnp.int32) % (n_pad - n))
        flat_idx = jnp.concatenate([flat_idx, pad_gather])
        dst = jnp.concatenate([dst, pad_dst.astype(jnp.int32)])
    g = e_pad // (_NW * _C)
    nb = g // _K

    all_t = _all_transforms(h, weight, bn=2000)
    all_t_flat = all_t.reshape(r * n, f_out)

    sc_fn = _make_sc_gather_scatter(n_pad, f_out, nb, _K, _C)
    partials = sc_fn(all_t_flat, flat_idx, dst)

    return _sum_partials(partials, n, bn=2000)
